# baseline (speedup 1.0000x reference)
import jax
import jax.numpy as jnp
from jax import lax
from jax.experimental import pallas as pl
from jax.experimental.pallas import tpu as pltpu
from jax.experimental.pallas import tpu_sc as plsc

M = 32
K = 8192
D = 64
N = 16384

NC = 2
NS = 16
NW = NC * NS
B = N * M
BPW = B // NW
G = 128
NG = BPW // G
L = 16


def _lookup_body(codes_hbm, table_hbm, out_hbm, idx_v, rows_v, sem0, sem1):
    wid = lax.axis_index("s") * NC + lax.axis_index("c")
    base = wid * BPW

    pltpu.sync_copy(codes_hbm.at[wid], idx_v)

    offs = [(lax.iota(jnp.int32, L) + j * L) * K for j in range(2)]

    def add_offsets(r):
        for j in range(G // L):
            sl = pl.ds(j * L, L)
            idx_v[r, sl] = idx_v[r, sl] + offs[j % 2]

    pl.loop(0, NG)(add_offsets)

    sems = (sem0, sem1)

    def gather(g, b):
        return pltpu.async_copy(table_hbm.at[idx_v.at[g]], rows_v.at[b], sems[b])

    gather(0, 0)
    gather(1, 1)

    def step(g0):
        for b in range(2):
            g = g0 + b
            pltpu.make_async_copy(table_hbm.at[idx_v.at[g]], rows_v.at[b],
                                  sems[b]).wait()
            pltpu.sync_copy(rows_v.at[b], out_hbm.at[pl.ds(base + g * G, G)])

            @pl.when(g + 2 < NG)
            def _():
                gather(g + 2, b)

    pl.loop(0, NG, step=2)(step)


@jax.jit
def kernel(codes, codebook):
    codes_i = codes.astype(jnp.int32).reshape(NW, NG, G)
    table = codebook.reshape(M * K, D)

    mesh = plsc.VectorSubcoreMesh(core_axis_name="c", subcore_axis_name="s")
    out = pl.kernel(
        _lookup_body,
        mesh=mesh,
        compiler_params=pltpu.CompilerParams(use_tc_tiling_on_sc=False),
        out_type=jax.ShapeDtypeStruct((B, D), jnp.float32),
        scratch_types=[
            pltpu.VMEM((NG, G), jnp.int32),
            pltpu.VMEM((2, G, D), jnp.float32),
            pltpu.SemaphoreType.DMA,
            pltpu.SemaphoreType.DMA,
        ],
    )(codes_i, table)
    return out.reshape(N, M, D)


# speedup vs baseline: 1.0375x; 1.0375x over previous
import jax
import jax.numpy as jnp
from jax import lax
from jax.experimental import pallas as pl
from jax.experimental.pallas import tpu as pltpu
from jax.experimental.pallas import tpu_sc as plsc

M = 32
K = 8192
D = 64
N = 16384

NC = 2
NS = 16
NW = NC * NS
B = N * M
BPW = B // NW
G = 128
NG = BPW // G
L = 16
NB = 8
FA = 4


def _lookup_body(codes_hbm, table_hbm, out_hbm, idx_v, rows_v, sem_g, sem_o):
    wid = lax.axis_index("s") * NC + lax.axis_index("c")
    base = wid * BPW

    pltpu.sync_copy(codes_hbm.at[wid], idx_v)

    offs = [(lax.iota(jnp.int32, L) + j * L) * K for j in range(2)]

    def add_offsets(r):
        for j in range(G // L):
            sl = pl.ds(j * L, L)
            idx_v[r, sl] = idx_v[r, sl] + offs[j % 2]

    pl.loop(0, NG)(add_offsets)

    def gather(g, b):
        pltpu.async_copy(table_hbm.at[idx_v.at[g]], rows_v.at[b], sem_g.at[b])

    def wait_gather(g, b):
        pltpu.make_async_copy(table_hbm.at[idx_v.at[g]], rows_v.at[b],
                              sem_g.at[b]).wait()

    def put(g, b):
        pltpu.async_copy(rows_v.at[b], out_hbm.at[pl.ds(base + g * G, G)],
                         sem_o.at[b])

    def wait_put(g, b):
        pltpu.make_async_copy(rows_v.at[b],
                              out_hbm.at[pl.ds(base + g * G, G)],
                              sem_o.at[b]).wait()

    for g in range(FA):
        gather(g, g)

    def step(g0):
        for b in range(NB):
            g = g0 + b
            wait_gather(g, b)
            put(g, b)
            gf = g + FA
            bf = (b + FA) % NB

            @pl.when(gf < NG)
            def _():
                @pl.when(gf >= NB)
                def _():
                    wait_put(gf - NB, bf)

                gather(gf, bf)

    pl.loop(0, NG, step=NB)(step)

    for b in range(NB):
        g = NG - NB + b
        wait_put(g, b)


@jax.jit
def kernel(codes, codebook):
    codes_i = codes.astype(jnp.int32).reshape(NW, NG, G)
    table = codebook.reshape(M * K, D)

    mesh = plsc.VectorSubcoreMesh(core_axis_name="c", subcore_axis_name="s")
    out = pl.kernel(
        _lookup_body,
        mesh=mesh,
        compiler_params=pltpu.CompilerParams(use_tc_tiling_on_sc=False),
        out_type=jax.ShapeDtypeStruct((B, D), jnp.float32),
        scratch_types=[
            pltpu.VMEM((NG, G), jnp.int32),
            pltpu.VMEM((NB, G, D), jnp.float32),
            pltpu.SemaphoreType.DMA((NB,)),
            pltpu.SemaphoreType.DMA((NB,)),
        ],
    )(codes_i, table)
    return out.reshape(N, M, D)


# speedup vs baseline: 1.2628x; 1.2171x over previous
import jax
import jax.numpy as jnp
from jax import lax
from jax.experimental import pallas as pl
from jax.experimental.pallas import tpu as pltpu
from jax.experimental.pallas import tpu_sc as plsc

M = 32
K = 8192
D = 64
N = 16384

NC = 2
NS = 16
MPC = M // NC
NPT = N // NS
G = 128
JPT = NPT // G
Q = 2
RPS = 256
SUB = NPT // RPS
L = 16


def _lookup_body(codes_hbm, table, out2, codes_v, idx_v, rows_v, sh0, sh1,
                 sem_pf, sem_g, sem_o):
    c = lax.axis_index("c")
    s = lax.axis_index("s")
    mbase = MPC * c
    nrow0 = s * NPT

    pltpu.sync_copy(codes_hbm.at[pl.ds(nrow0, NPT), pl.ds(mbase, MPC)],
                    codes_v)

    shs = (sh0, sh1)

    def pf_copy(ph, p):
        m = mbase + ph
        return pltpu.make_async_copy(
            table.at[pl.ds(m * K + s * (K // NS), K // NS)],
            shs[p].at[pl.ds(s * (K // NS), K // NS)],
            sem_pf.at[p])

    def out_copy(ph, u):
        m = mbase + ph
        q = u % Q
        return pltpu.make_async_copy(
            rows_v.at[q],
            out2.at[pl.ds(nrow0 + u * RPS, RPS), pl.ds(m * D, D)],
            sem_o.at[q])

    def extract(ph, p):
        col = jnp.full((L,), ph, jnp.int32)
        for i in range(NPT // L):
            rows = lax.iota(jnp.int32, L) + i * L
            vals = plsc.load_gather(codes_v, [rows, col])
            idx_v[p, i // (G // L), pl.ds((i % (G // L)) * L, L)] = vals

    pf_copy(0, 0).start()
    pf_copy(1, 1).start()

    def phase(ph, p):
        extract(ph, p)
        pf_copy(ph, p).wait()
        plsc.subcore_barrier()

        for u in range(SUB):
            q = u % Q
            if u >= Q:
                out_copy(ph, u - Q).wait()
            else:
                @pl.when(ph > 0)
                def _():
                    out_copy(ph - 1, u - Q + SUB).wait()

            for k in range(RPS // G):
                pltpu.async_copy(
                    shs[p].at[idx_v.at[p, u * (RPS // G) + k]],
                    rows_v.at[q, pl.ds(k * G, G)],
                    sem_g.at[q])
            for k in range(RPS // G):
                pltpu.make_async_copy(
                    shs[p].at[idx_v.at[p, u * (RPS // G) + k]],
                    rows_v.at[q, pl.ds(k * G, G)],
                    sem_g.at[q]).wait()
            out_copy(ph, u).start()

        plsc.subcore_barrier()

        @pl.when(ph + 2 < MPC)
        def _():
            pf_copy(ph + 2, p).start()

    def step(ph0):
        for parity in range(2):
            phase(ph0 + parity, parity)

    pl.loop(0, MPC, step=2)(step)

    for u in range(SUB - Q, SUB):
        out_copy(MPC - 1, u).wait()


@jax.jit
def kernel(codes, codebook):
    codes_i = codes.astype(jnp.int32)
    table = codebook.reshape(M * K, D)

    mesh = plsc.VectorSubcoreMesh(core_axis_name="c", subcore_axis_name="s")
    out = pl.kernel(
        _lookup_body,
        mesh=mesh,
        compiler_params=pltpu.CompilerParams(use_tc_tiling_on_sc=False,
                                             needs_layout_passes=False),
        out_type=jax.ShapeDtypeStruct((N, M * D), jnp.float32),
        scratch_types=[
            pltpu.VMEM((NPT, MPC), jnp.int32),
            pltpu.VMEM((2, JPT, G), jnp.int32),
            pltpu.VMEM((Q, RPS, D), jnp.float32),
            pltpu.VMEM_SHARED((K, D), jnp.float32),
            pltpu.VMEM_SHARED((K, D), jnp.float32),
            pltpu.SemaphoreType.DMA((2,)),
            pltpu.SemaphoreType.DMA((Q,)),
            pltpu.SemaphoreType.DMA((Q,)),
        ],
    )(codes_i, table)
    return out.reshape(N, M, D)


# speedup vs baseline: 1.5421x; 1.2212x over previous
import jax
import jax.numpy as jnp
from jax import lax
from jax.experimental import pallas as pl
from jax.experimental.pallas import tpu as pltpu
from jax.experimental.pallas import tpu_sc as plsc

M = 32
K = 8192
D = 64
N = 16384

NC = 2
NS = 16
MPC = M // NC
MG = 8
DG = 8
NCH = 2048
NCHN = N // NCH
L = 16


def _lookup_body(codes_t, cb_t, out, stage_v, tab_v, idx_v, out_v, codes_sh,
                 sem_st, sem_tab, sem_idx, sem_out):
    c = lax.axis_index("c")
    s = lax.axis_index("s")
    mg = s // DG
    dg = s % DG
    mrow0 = MPC * c

    def tab_copy(mi):
        m = mrow0 + mg * MG + mi
        return pltpu.make_async_copy(
            cb_t.at[m, pl.ds(dg * DG, DG)], tab_v, sem_tab)

    tab_copy(0).start()

    SB = 1024
    for r in range(2):
        b = 2 * s + r
        st_mg = b // 16
        st_nb = b % 16

        def st_copy():
            return pltpu.make_async_copy(
                codes_t.at[pl.ds(mrow0 + st_mg * MG, MG),
                           pl.ds(st_nb * SB, SB)],
                stage_v, sem_st)

        st_copy().start()
        st_copy().wait()
        for mr in range(MG):
            pltpu.async_copy(
                stage_v.at[mr],
                codes_sh.at[st_mg * MG + mr, pl.ds(st_nb * SB, SB)],
                sem_st)
        for mr in range(MG):
            pltpu.make_async_copy(
                stage_v.at[mr],
                codes_sh.at[st_mg * MG + mr, pl.ds(st_nb * SB, SB)],
                sem_st).wait()
    plsc.subcore_barrier()

    def idx_copy(mi, ch, p):
        return pltpu.make_async_copy(
            codes_sh.at[mg * MG + mi, pl.ds(ch * NCH, NCH)],
            idx_v.at[p], sem_idx.at[p])

    def out_copy(mi, ch, p):
        m = mrow0 + mg * MG + mi
        return pltpu.make_async_copy(
            out_v.at[p],
            out.at[pl.ds(m * D + dg * DG, DG), pl.ds(ch * NCH, NCH)],
            sem_out.at[p])

    idx_copy(0, 0, 0).start()

    MT = MPC // NC
    dvecs = [jnp.full((L,), d, jnp.int32) for d in range(DG)]

    def idx_wait(p):
        idx_copy(0, 0, p).wait()

    def out_wait(p):
        out_copy(0, 0, p).wait()

    def chunk(mi, ch, p, first, idx_pf, idx_pf_next_m):
        idx_wait(p)
        idx_pf(p)
        idx_pf_next_m(p)

        @pl.when(jnp.logical_not(first))
        def _():
            out_wait(p)

        def gather(i):
            for u in range(2):
                iv = idx_v[p, pl.ds((2 * i + u) * L, L)]
                for d in range(DG):
                    vals = plsc.load_gather(tab_v, [dvecs[d], iv])
                    out_v[p, d, pl.ds((2 * i + u) * L, L)] = vals

        pl.loop(0, NCH // L // 2)(gather)
        out_copy(mi, ch, p).start()

    def m_step(mi):
        tab_copy(0).wait()

        def ch_step(chv):
            def pf_a(p):
                idx_copy(mi, chv + 1, 1 - p).start()

            def pf_none(p):
                pass

            def pf_b(p):
                @pl.when(chv + 2 < NCHN)
                def _():
                    idx_copy(mi, chv + 2, 1 - p).start()

            def pf_b2(p):
                @pl.when(jnp.logical_and(chv + 2 >= NCHN, mi + 1 < MT))
                def _():
                    idx_copy(mi + 1, 0, 1 - p).start()

            first0 = jnp.logical_and(mi == 0, chv == 0)
            chunk(mi, chv, 0, first0, pf_a, pf_none)
            chunk(mi, chv + 1, 1, first0, pf_b, pf_b2)

        pl.loop(0, NCHN, step=2)(ch_step)

        @pl.when(mi + 1 < MT)
        def _():
            tab_copy(mi + 1).start()

    pl.loop(0, MT)(m_step)

    out_wait(0)
    out_wait(1)


@jax.jit
def kernel(codes, codebook):
    codes_t = codes.astype(jnp.int32).T
    cb_t = codebook.transpose(0, 2, 1)

    mesh = plsc.VectorSubcoreMesh(core_axis_name="c", subcore_axis_name="s")
    out = pl.kernel(
        _lookup_body,
        mesh=mesh,
        compiler_params=pltpu.CompilerParams(use_tc_tiling_on_sc=True,
                                             needs_layout_passes=False),
        out_type=jax.ShapeDtypeStruct((M * D, N), jnp.float32),
        scratch_types=[
            pltpu.VMEM((MG, 1024), jnp.int32),
            pltpu.VMEM((DG, K), jnp.float32),
            pltpu.VMEM((2, NCH), jnp.int32),
            pltpu.VMEM((2, DG, NCH), jnp.float32),
            pltpu.VMEM_SHARED((MPC, N), jnp.int32),
            pltpu.SemaphoreType.DMA,
            pltpu.SemaphoreType.DMA,
            pltpu.SemaphoreType.DMA((2,)),
            pltpu.SemaphoreType.DMA((2,)),
        ],
    )(codes_t, cb_t)
    return out.reshape(M, D, N).transpose(2, 0, 1)
